# baseline (speedup 1.0000x reference)
import functools

import jax
import jax.numpy as jnp
from jax.experimental import pallas as pl
from jax.experimental.pallas import tpu as pltpu

_NB = 512


def _chamfer_kernel(nblocks, c1, c2, pred_ref, tgt_ref, loss_ref, colmin_ref):
    b = pl.program_id(0)
    nb = pl.program_id(1)

    p = pred_ref[0]
    t = tgt_ref[0]
    pb = p.astype(jnp.bfloat16).astype(jnp.float32)
    tb = t.astype(jnp.bfloat16).astype(jnp.float32)
    p2 = jnp.sum(p * p, axis=1, keepdims=True)
    t2 = jnp.sum(t * t, axis=0, keepdims=True)
    cross = (pb[:, 0:1] * tb[0:1, :]
             + pb[:, 1:2] * tb[1:2, :]
             + pb[:, 2:3] * tb[2:3, :])
    d = jnp.maximum(p2 + t2 - 2.0 * cross, 0.0)

    @pl.when(jnp.logical_and(b == 0, nb == 0))
    def _():
        loss_ref[...] = jnp.zeros((1, 1), jnp.float32)

    row_sum = jnp.sum(jnp.min(d, axis=1, keepdims=True), axis=0, keepdims=True)
    loss_ref[...] += row_sum * c1

    colmin = jnp.min(d, axis=0, keepdims=True)

    @pl.when(nb == 0)
    def _():
        colmin_ref[...] = colmin

    @pl.when(nb != 0)
    def _():
        colmin_ref[...] = jnp.minimum(colmin_ref[...], colmin)

    @pl.when(nb == nblocks - 1)
    def _():
        col_sum = jnp.sum(colmin_ref[...], axis=1, keepdims=True)
        loss_ref[...] += col_sum * c2


def kernel(pred, target):
    B, N, D = pred.shape
    _, M, _ = target.shape
    tgt = jnp.swapaxes(target, 1, 2)
    nblocks = N // _NB
    c1 = 0.5 / (B * N)
    c2 = 0.5 / (B * M)
    loss = pl.pallas_call(
        functools.partial(_chamfer_kernel, nblocks, c1, c2),
        grid=(B, nblocks),
        in_specs=[
            pl.BlockSpec((1, _NB, D), lambda b, nb: (b, nb, 0)),
            pl.BlockSpec((1, D, M), lambda b, nb: (b, 0, 0)),
        ],
        out_specs=pl.BlockSpec((1, 1), lambda b, nb: (0, 0)),
        out_shape=jax.ShapeDtypeStruct((1, 1), jnp.float32),
        scratch_shapes=[pltpu.VMEM((1, M), jnp.float32)],
    )(pred, tgt)
    return loss[0, 0]


# speedup vs baseline: 1.4882x; 1.4882x over previous
import functools

import jax
import jax.numpy as jnp
from jax.experimental import pallas as pl
from jax.experimental.pallas import tpu as pltpu

_NB = 512
_K = 8


def _chamfer_kernel(nblocks, c1, c2, lhs_ref, rhs_ref, loss_ref, colmin_ref):
    b = pl.program_id(0)
    nb = pl.program_id(1)

    f = jax.lax.dot_general(
        lhs_ref[0], rhs_ref[0], (((1,), (0,)), ((), ())),
        preferred_element_type=jnp.float32)

    @pl.when(jnp.logical_and(b == 0, nb == 0))
    def _():
        loss_ref[...] = jnp.zeros((1, 1), jnp.float32)

    rowmin = jnp.min(f, axis=1, keepdims=True)
    row_sum = jnp.sum(jnp.maximum(rowmin, 0.0), axis=0, keepdims=True)
    loss_ref[...] += row_sum * c1

    colmin = jnp.min(f, axis=0, keepdims=True)

    @pl.when(nb == 0)
    def _():
        colmin_ref[...] = colmin

    @pl.when(nb != 0)
    def _():
        colmin_ref[...] = jnp.minimum(colmin_ref[...], colmin)

    @pl.when(nb == nblocks - 1)
    def _():
        col_sum = jnp.sum(jnp.maximum(colmin_ref[...], 0.0), axis=1,
                          keepdims=True)
        loss_ref[...] += col_sum * c2


def _augment(pts, norm_side):
    sq = jnp.sum(pts * pts, axis=2, keepdims=True)
    hi = sq.astype(jnp.bfloat16).astype(jnp.float32)
    lo = sq - hi
    one = jnp.ones_like(sq)
    zero = jnp.zeros_like(sq)
    if norm_side == "lhs":
        cols = [-2.0 * pts, hi, lo, one, one, zero]
    else:
        cols = [pts, one, one, hi, lo, zero]
    return jnp.concatenate(cols, axis=2).astype(jnp.bfloat16)


def kernel(pred, target):
    B, N, D = pred.shape
    M = target.shape[1]
    lhs = _augment(pred, "lhs")
    rhs = jnp.swapaxes(_augment(target, "rhs"), 1, 2)
    nblocks = N // _NB
    c1 = 0.5 / (B * N)
    c2 = 0.5 / (B * M)
    loss = pl.pallas_call(
        functools.partial(_chamfer_kernel, nblocks, c1, c2),
        grid=(B, nblocks),
        in_specs=[
            pl.BlockSpec((1, _NB, _K), lambda b, nb: (b, nb, 0)),
            pl.BlockSpec((1, _K, M), lambda b, nb: (b, 0, 0)),
        ],
        out_specs=pl.BlockSpec((1, 1), lambda b, nb: (0, 0)),
        out_shape=jax.ShapeDtypeStruct((1, 1), jnp.float32),
        scratch_shapes=[pltpu.VMEM((1, M), jnp.float32)],
    )(lhs, rhs)
    return loss[0, 0]


# speedup vs baseline: 1.7992x; 1.2090x over previous
import functools

import jax
import jax.numpy as jnp
from jax.experimental import pallas as pl
from jax.experimental.pallas import tpu as pltpu

_NB = 512
_K = 8


_NC = 128


def _chamfer_kernel(nblocks, c1, c2, lhs_ref, rhs_ref, loss_ref, colmin_ref):
    b = pl.program_id(0)
    nb = pl.program_id(1)

    lhs = lhs_ref[0]
    rhs = rhs_ref[0]

    @pl.when(jnp.logical_and(b == 0, nb == 0))
    def _():
        loss_ref[...] = jnp.zeros((1, 1), jnp.float32)

    row_sum = None
    colmin = None
    for c in range(_NB // _NC):
        f = jax.lax.dot_general(
            lhs[c * _NC:(c + 1) * _NC], rhs, (((1,), (0,)), ((), ())),
            preferred_element_type=jnp.float32)
        rowmin = jnp.min(f, axis=1, keepdims=True)
        rs = jnp.sum(jnp.maximum(rowmin, 0.0), axis=0, keepdims=True)
        row_sum = rs if row_sum is None else row_sum + rs
        cm = jnp.min(f, axis=0, keepdims=True)
        colmin = cm if colmin is None else jnp.minimum(colmin, cm)

    loss_ref[...] += row_sum * c1


    @pl.when(nb == 0)
    def _():
        colmin_ref[...] = colmin

    @pl.when(nb != 0)
    def _():
        colmin_ref[...] = jnp.minimum(colmin_ref[...], colmin)

    @pl.when(nb == nblocks - 1)
    def _():
        col_sum = jnp.sum(jnp.maximum(colmin_ref[...], 0.0), axis=1,
                          keepdims=True)
        loss_ref[...] += col_sum * c2


def _augment(pts, norm_side):
    sq = jnp.sum(pts * pts, axis=2, keepdims=True)
    hi = sq.astype(jnp.bfloat16).astype(jnp.float32)
    lo = sq - hi
    one = jnp.ones_like(sq)
    zero = jnp.zeros_like(sq)
    if norm_side == "lhs":
        cols = [-2.0 * pts, hi, lo, one, one, zero]
    else:
        cols = [pts, one, one, hi, lo, zero]
    return jnp.concatenate(cols, axis=2).astype(jnp.bfloat16)


def kernel(pred, target):
    B, N, D = pred.shape
    M = target.shape[1]
    lhs = _augment(pred, "lhs")
    rhs = jnp.swapaxes(_augment(target, "rhs"), 1, 2)
    nblocks = N // _NB
    c1 = 0.5 / (B * N)
    c2 = 0.5 / (B * M)
    loss = pl.pallas_call(
        functools.partial(_chamfer_kernel, nblocks, c1, c2),
        grid=(B, nblocks),
        in_specs=[
            pl.BlockSpec((1, _NB, _K), lambda b, nb: (b, nb, 0)),
            pl.BlockSpec((1, _K, M), lambda b, nb: (b, 0, 0)),
        ],
        out_specs=pl.BlockSpec((1, 1), lambda b, nb: (0, 0)),
        out_shape=jax.ShapeDtypeStruct((1, 1), jnp.float32),
        scratch_shapes=[pltpu.VMEM((1, M), jnp.float32)],
    )(lhs, rhs)
    return loss[0, 0]


# speedup vs baseline: 1.9645x; 1.0919x over previous
import functools

import jax
import jax.numpy as jnp
from jax.experimental import pallas as pl
from jax.experimental.pallas import tpu as pltpu

_K = 8
_NC = 128


def _chamfer_kernel(c1, c2, lhs_ref, rhs_ref, loss_ref):
    b = pl.program_id(0)
    lhs = lhs_ref[0]
    rhs = rhs_ref[0]
    N = lhs.shape[0]
    M = rhs.shape[1]

    @pl.when(b == 0)
    def _():
        loss_ref[...] = jnp.zeros((1, 1), jnp.float32)

    row_sum = None
    colmin8 = None
    for c in range(N // _NC):
        f = jax.lax.dot_general(
            lhs[c * _NC:(c + 1) * _NC], rhs, (((1,), (0,)), ((), ())),
            preferred_element_type=jnp.float32)
        rowmin = jnp.min(f, axis=1, keepdims=True)
        rs = jnp.sum(jnp.maximum(rowmin, 0.0), axis=0, keepdims=True)
        row_sum = rs if row_sum is None else row_sum + rs
        cm8 = jnp.min(f.reshape(_NC // 8, 8, M), axis=0)
        colmin8 = cm8 if colmin8 is None else jnp.minimum(colmin8, cm8)

    colmin = jnp.min(jnp.maximum(colmin8, 0.0), axis=0, keepdims=True)
    col_sum = jnp.sum(colmin, axis=1, keepdims=True)
    loss_ref[...] += row_sum * c1 + col_sum * c2


def _augment(pts, norm_side):
    sq = jnp.sum(pts * pts, axis=2, keepdims=True)
    hi = sq.astype(jnp.bfloat16).astype(jnp.float32)
    lo = sq - hi
    one = jnp.ones_like(sq)
    zero = jnp.zeros_like(sq)
    if norm_side == "lhs":
        cols = [-2.0 * pts, hi, lo, one, one, zero]
    else:
        cols = [pts, one, one, hi, lo, zero]
    return jnp.concatenate(cols, axis=2).astype(jnp.bfloat16)


def kernel(pred, target):
    B, N, D = pred.shape
    M = target.shape[1]
    lhs = _augment(pred, "lhs")
    rhs = jnp.swapaxes(_augment(target, "rhs"), 1, 2)
    c1 = 0.5 / (B * N)
    c2 = 0.5 / (B * M)
    loss = pl.pallas_call(
        functools.partial(_chamfer_kernel, c1, c2),
        grid=(B,),
        in_specs=[
            pl.BlockSpec((1, N, _K), lambda b: (b, 0, 0)),
            pl.BlockSpec((1, _K, M), lambda b: (b, 0, 0)),
        ],
        out_specs=pl.BlockSpec((1, 1), lambda b: (0, 0)),
        out_shape=jax.ShapeDtypeStruct((1, 1), jnp.float32),
    )(lhs, rhs)
    return loss[0, 0]


# speedup vs baseline: 3.6756x; 1.8710x over previous
import functools

import jax
import jax.numpy as jnp
from jax.experimental import pallas as pl
from jax.experimental.pallas import tpu as pltpu

_NC = 128


def _chamfer_kernel(c1, c2, pred_ref, tgt_ref, loss_ref):
    b = pl.program_id(0)
    p = pred_ref[0]
    t = tgt_ref[0]
    N = p.shape[0]
    M = t.shape[1]

    @pl.when(b == 0)
    def _():
        loss_ref[...] = jnp.zeros((1, 1), jnp.float32)

    p2 = jnp.sum(p * p, axis=1, keepdims=True)
    p2h = p2.astype(jnp.bfloat16).astype(jnp.float32)
    p2l = p2 - p2h
    ones_n = jnp.ones((N, 1), jnp.float32)
    zero_n = jnp.zeros((N, 1), jnp.float32)
    lhs = jnp.concatenate(
        [-2.0 * p, p2h, p2l, ones_n, ones_n, zero_n],
        axis=1).astype(jnp.bfloat16)

    t2 = jnp.sum(t * t, axis=0, keepdims=True)
    t2h = t2.astype(jnp.bfloat16).astype(jnp.float32)
    t2l = t2 - t2h
    ones_m = jnp.ones((1, M), jnp.float32)
    zero_m = jnp.zeros((1, M), jnp.float32)
    rhs = jnp.concatenate(
        [t, ones_m, ones_m, t2h, t2l, zero_m],
        axis=0).astype(jnp.bfloat16)

    row_sum = None
    colmin8 = None
    for c in range(N // _NC):
        f = jax.lax.dot_general(
            lhs[c * _NC:(c + 1) * _NC], rhs, (((1,), (0,)), ((), ())),
            preferred_element_type=jnp.float32)
        rowmin = jnp.min(f, axis=1, keepdims=True)
        rs = jnp.sum(jnp.maximum(rowmin, 0.0), axis=0, keepdims=True)
        row_sum = rs if row_sum is None else row_sum + rs
        cm8 = jnp.min(f.reshape(_NC // 8, 8, M), axis=0)
        colmin8 = cm8 if colmin8 is None else jnp.minimum(colmin8, cm8)

    colmin = jnp.min(jnp.maximum(colmin8, 0.0), axis=0, keepdims=True)
    col_sum = jnp.sum(colmin, axis=1, keepdims=True)
    loss_ref[...] += row_sum * c1 + col_sum * c2


def kernel(pred, target):
    B, N, D = pred.shape
    M = target.shape[1]
    tgt = jnp.swapaxes(target, 1, 2)
    c1 = 0.5 / (B * N)
    c2 = 0.5 / (B * M)
    loss = pl.pallas_call(
        functools.partial(_chamfer_kernel, c1, c2),
        grid=(B,),
        in_specs=[
            pl.BlockSpec((1, N, D), lambda b: (b, 0, 0)),
            pl.BlockSpec((1, D, M), lambda b: (b, 0, 0)),
        ],
        out_specs=pl.BlockSpec((1, 1), lambda b: (0, 0)),
        out_shape=jax.ShapeDtypeStruct((1, 1), jnp.float32),
    )(pred, tgt)
    return loss[0, 0]
